# baseline (speedup 1.0000x reference)
import functools

import jax
import jax.numpy as jnp
from jax import lax
from jax.experimental import pallas as pl
from jax.experimental.pallas import tpu as pltpu
from jax.experimental.pallas import tpu_sc as plsc

N = 10000
D = 128
E = 320000
NPAD = 10240
NC = 2
NS = 16
NW = NC * NS
EPW = 10080
EPADTOT = NW * EPW
C = 48
NCHUNK = EPW // C
SINK = N
RES_W = 0.1


def _snorm(W):
    u = jnp.ones((W.shape[0],), dtype=W.dtype) / jnp.sqrt(jnp.float32(W.shape[0]))
    v = None
    for _ in range(3):
        v = W.T @ u
        v = v / (jnp.linalg.norm(v) + 1e-12)
        u = W @ v
        u = u / (jnp.linalg.norm(u) + 1e-12)
    sigma = u @ (W @ v)
    return W / (sigma + 1e-12)


def _mm_body(x_ref, w_ref, p_ref, b_ref):
    y = jax.lax.dot_general(
        x_ref[...], w_ref[...], (((1,), (0,)), ((), ())),
        preferred_element_type=jnp.float32,
        precision=jax.lax.Precision.HIGHEST)
    p_ref[...] = y[:, :2 * D]
    b_ref[...] = y[:, 2 * D:]


def _precompute(xp, wcat):
    bn = 512
    return pl.pallas_call(
        _mm_body,
        grid=(NPAD // bn,),
        in_specs=[pl.BlockSpec((bn, D), lambda i: (i, 0)),
                  pl.BlockSpec((D, 3 * D), lambda i: (0, 0))],
        out_specs=[pl.BlockSpec((bn, 2 * D), lambda i: (i, 0)),
                   pl.BlockSpec((bn, D), lambda i: (i, 0))],
        out_shape=[jax.ShapeDtypeStruct((NPAD, 2 * D), jnp.float32),
                   jax.ShapeDtypeStruct((NPAD, D), jnp.float32)],
    )(xp, wcat)


def _sc_edges(P, Bm, eij, b1, w2, b2v):
    mesh = plsc.VectorSubcoreMesh(core_axis_name="c", subcore_axis_name="s")

    @functools.partial(
        pl.kernel,
        mesh=mesh,
        compiler_params=pltpu.CompilerParams(use_tc_tiling_on_sc=False,
                                             needs_layout_passes=False),
        out_type=jax.ShapeDtypeStruct((NC, NPAD, D), jnp.float32),
        scratch_types=[
            pltpu.VMEM((C, 2 * D), jnp.float32),
            pltpu.VMEM((C, 2 * D), jnp.float32),
            pltpu.VMEM((C, D), jnp.float32),
            pltpu.VMEM((C, D), jnp.float32),
            pltpu.VMEM((2, C), jnp.int32),
            pltpu.VMEM((2, C), jnp.int32),
            pltpu.VMEM((C,), jnp.int32),
            pltpu.VMEM((C,), jnp.int32),
            pltpu.VMEM((D,), jnp.float32),
            pltpu.VMEM((D,), jnp.float32),
            pltpu.VMEM((16,), jnp.float32),
            pltpu.VMEM_SHARED((NPAD, D), jnp.float32),
            pltpu.SemaphoreType.DMA,
            pltpu.SemaphoreType.DMA,
            pltpu.SemaphoreType.DMA,
            pltpu.SemaphoreType.DMA,
            pltpu.SemaphoreType.DMA,
            pltpu.SemaphoreType.DMA,
            pltpu.SemaphoreType.DMA,
            pltpu.SemaphoreType.DMA,
        ],
    )
    def k(P_hbm, Bm_hbm, eij_hbm, b1_hbm, w2_hbm, b2_hbm, out_hbm,
          p0, p1, bm0, bm1, ij0, ij1, dc0, dc1, b1v, w2v, b2vv, acc,
          sp0, sp1, sb0, sb1, si0, si1, ss0, ss1):
        pb = (p0, p1)
        bb = (bm0, bm1)
        mb = (bm0, bm1)
        ijb = (ij0, ij1)
        dcb = (dc0, dc1)
        sem_p = (sp0, sp1)
        sem_b = (sb0, sb1)
        sem_i = (si0, si1)
        sem_s = (ss0, ss1)

        cid = lax.axis_index("c")
        sid = lax.axis_index("s")
        wid = sid * NC + cid
        ebase = wid * EPW

        pltpu.sync_copy(b1_hbm, b1v)
        pltpu.sync_copy(w2_hbm, w2v)
        pltpu.sync_copy(b2_hbm, b2vv)

        zeros16 = jnp.zeros((16,), jnp.float32)

        def zrow(r, carry):
            for kk in range(D // 16):
                bm0[r, pl.ds(kk * 16, 16)] = zeros16
            return carry

        lax.fori_loop(0, C, zrow, 0)
        rows_per_tile = NPAD // NS
        rbase = sid * rows_per_tile
        for z in range(rows_per_tile // C):
            pltpu.sync_copy(bm0, acc.at[pl.ds(rbase + z * C, C)])
        rem = rows_per_tile % C
        if rem:
            pltpu.sync_copy(bm0.at[pl.ds(0, rem)],
                            acc.at[pl.ds(rbase + (rows_per_tile // C) * C, rem)])
        plsc.subcore_barrier()

        b2vec = b2vv[...]
        b2s = b2vec[0]
        zero16 = jnp.zeros((16,), jnp.float32)
        b1r = [b1v[pl.ds(kk * 16, 16)] for kk in range(D // 16)]
        w2r = [w2v[pl.ds(kk * 16, 16)] for kk in range(D // 16)]

        def fetch_idx(ci, par):
            return pltpu.async_copy(
                eij_hbm.at[:, pl.ds(ebase + ci * C, C)], ijb[par], sem_i[par])

        def copy_scatter_idx(par):
            for kk in range(C // 16):
                dcb[par][pl.ds(kk * 16, 16)] = ijb[par][1, pl.ds(kk * 16, 16)]

        def issue_gathers(par):
            pltpu.async_copy(P_hbm.at[ijb[par].at[0]], pb[par], sem_p[par])
            pltpu.async_copy(Bm_hbm.at[ijb[par].at[1]], bb[par], sem_b[par])

        def wait_gathers(par):
            pltpu.make_async_copy(P_hbm.at[pl.ds(0, C)], pb[par], sem_p[par]).wait()
            pltpu.make_async_copy(Bm_hbm.at[pl.ds(0, C)], bb[par], sem_b[par]).wait()

        def wait_idx(par):
            pltpu.make_async_copy(
                eij_hbm.at[:, pl.ds(0, C)], ijb[par], sem_i[par]).wait()

        def wait_scatter(par):
            pltpu.make_async_copy(mb[par], acc.at[pl.ds(0, C)], sem_s[par]).wait()

        def compute(par):
            msg = mb[par]
            p_rows = pb[par]
            b_rows = bb[par]

            def group_body(g, gcarry):
                e0 = g * 16
                for l in range(16):
                    e = e0 + l
                    t = None
                    for kk in range(D // 16):
                        a = p_rows[e, pl.ds(kk * 16, 16)]
                        b = b_rows[e, pl.ds(kk * 16, 16)]
                        h = jnp.maximum(a + b + b1r[kk], 0.0)
                        term = h * w2r[kk]
                        t = term if t is None else t + term
                    s = jnp.sum(t) + b2s
                    av = zero16 + s
                    al_vec = 1.0 / (1.0 + jnp.exp(-av))
                    for kk in range(D // 16):
                        msg[e, pl.ds(kk * 16, 16)] = (
                            al_vec * p_rows[e, pl.ds(D + kk * 16, 16)])
                return gcarry

            lax.fori_loop(0, C // 16, group_body, 0)

        def issue_scatter(par):
            pltpu.async_copy(mb[par], acc.at[dcb[par]], sem_s[par], add=True)

        fetch_idx(0, 0).wait()
        copy_scatter_idx(0)
        issue_gathers(0)
        fetch_idx(1, 1)

        wait_gathers(0)
        wait_idx(1)
        copy_scatter_idx(1)
        issue_gathers(1)
        fetch_idx(2, 0)
        compute(0)
        issue_scatter(0)

        def step(i, par):
            nxt_par = 1 - par
            wait_gathers(par)
            wait_scatter(nxt_par)
            wait_idx(nxt_par)
            copy_scatter_idx(nxt_par)
            issue_gathers(nxt_par)

            @pl.when(i + 2 < NCHUNK)
            def _():
                fetch_idx(i + 2, par)

            compute(par)
            issue_scatter(par)

        def pair_body(t, carry):
            step(2 * t + 1, 1)
            step(2 * t + 2, 0)
            return carry

        lax.fori_loop(0, (NCHUNK - 2) // 2, pair_body, 0)

        wait_gathers(1)
        wait_scatter(0)
        compute(1)
        issue_scatter(1)
        wait_scatter(1)

        plsc.subcore_barrier()
        out_rows = NPAD // NS
        pltpu.sync_copy(acc.at[pl.ds(sid * out_rows, out_rows)],
                        out_hbm.at[cid, pl.ds(sid * out_rows, out_rows)])

    return k(P, Bm, eij, b1, w2, b2v)


def _fin_body(p_ref, x_ref, bg_ref, o_ref):
    o_ref[...] = (p_ref[0] + p_ref[1] + (1.0 - RES_W) * bg_ref[...]
                  + RES_W * x_ref[...])


def _final(parts, x, bg2):
    bn = 1000
    return pl.pallas_call(
        _fin_body,
        grid=(N // bn,),
        in_specs=[pl.BlockSpec((NC, bn, D), lambda i: (0, i, 0)),
                  pl.BlockSpec((bn, D), lambda i: (i, 0)),
                  pl.BlockSpec((1, D), lambda i: (0, 0))],
        out_specs=pl.BlockSpec((bn, D), lambda i: (i, 0)),
        out_shape=jax.ShapeDtypeStruct((N, D), jnp.float32),
    )(parts, x, bg2)


def kernel(x, edge_index, W1, b1, W2, b2, Wg, bg):
    W1n = _snorm(W1)
    W2n = _snorm(W2)
    Wgn = _snorm(Wg)
    wcat = jnp.concatenate(
        [W1n[:D, :], (1.0 - RES_W) * Wgn, W1n[D:, :]], axis=1)
    xp = jnp.concatenate([x, jnp.zeros((NPAD - N, D), x.dtype)], axis=0)
    P, Bm = _precompute(xp, wcat)
    srcp = jnp.concatenate(
        [edge_index[0], jnp.zeros((EPADTOT - E,), jnp.int32)])
    dstp = jnp.concatenate(
        [edge_index[1], jnp.full((EPADTOT - E,), SINK, jnp.int32)])
    eij = jnp.stack([srcp, dstp])
    b2v = jnp.broadcast_to(b2.astype(jnp.float32), (16,))
    parts = _sc_edges(P, Bm, eij, b1, W2n[:, 0], b2v)
    return _final(parts, x, bg.reshape(1, D))


# speedup vs baseline: 1.9652x; 1.9652x over previous
import functools

import jax
import jax.numpy as jnp
from jax import lax
from jax.experimental import pallas as pl
from jax.experimental.pallas import tpu as pltpu
from jax.experimental.pallas import tpu_sc as plsc

N = 10000
D = 128
E = 320000
NPAD = 10240
NC = 2
NS = 16
NW = NC * NS
EPW = 10080
EPADTOT = NW * EPW
C = 48
NCHUNK = EPW // C
SINK = N
RES_W = 0.1


def _snorm(W):
    u = jnp.ones((W.shape[0],), dtype=W.dtype) / jnp.sqrt(jnp.float32(W.shape[0]))
    v = None
    for _ in range(3):
        v = W.T @ u
        v = v / (jnp.linalg.norm(v) + 1e-12)
        u = W @ v
        u = u / (jnp.linalg.norm(u) + 1e-12)
    sigma = u @ (W @ v)
    return W / (sigma + 1e-12)


def _mm_body(x_ref, w_ref, u_ref):
    y = jax.lax.dot_general(
        x_ref[...], w_ref[...], (((1,), (0,)), ((), ())),
        preferred_element_type=jnp.float32,
        precision=jax.lax.Precision.HIGHEST)
    u_ref[0] = y[:, :D]
    u_ref[1] = y[:, 2 * D:]
    u_ref[2] = y[:, D:2 * D]


def _precompute(xp, wcat):
    bn = 512
    return pl.pallas_call(
        _mm_body,
        grid=(NPAD // bn,),
        in_specs=[pl.BlockSpec((bn, D), lambda i: (i, 0)),
                  pl.BlockSpec((D, 3 * D), lambda i: (0, 0))],
        out_specs=pl.BlockSpec((3, bn, D), lambda i: (0, i, 0)),
        out_shape=jax.ShapeDtypeStruct((3, NPAD, D), jnp.float32),
    )(xp, wcat)


def _sc_edges(U, eij, b1, w2, b2v):
    mesh = plsc.VectorSubcoreMesh(core_axis_name="c", subcore_axis_name="s")

    @functools.partial(
        pl.kernel,
        mesh=mesh,
        compiler_params=pltpu.CompilerParams(use_tc_tiling_on_sc=False,
                                             needs_layout_passes=False),
        out_type=jax.ShapeDtypeStruct((NC, NPAD, D), jnp.float32),
        scratch_types=[
            pltpu.VMEM((3 * C, D), jnp.float32),
            pltpu.VMEM((3 * C, D), jnp.float32),
            pltpu.VMEM((2, C), jnp.int32),
            pltpu.VMEM((2, C), jnp.int32),
            pltpu.VMEM((3 * C,), jnp.int32),
            pltpu.VMEM((3 * C,), jnp.int32),
            pltpu.VMEM((C,), jnp.int32),
            pltpu.VMEM((C,), jnp.int32),
            pltpu.VMEM((D,), jnp.float32),
            pltpu.VMEM((D,), jnp.float32),
            pltpu.VMEM((16,), jnp.float32),
            pltpu.VMEM_SHARED((NPAD, D), jnp.float32),
            pltpu.SemaphoreType.DMA,
            pltpu.SemaphoreType.DMA,
            pltpu.SemaphoreType.DMA,
            pltpu.SemaphoreType.DMA,
            pltpu.SemaphoreType.DMA,
            pltpu.SemaphoreType.DMA,
        ],
    )
    def k(U_hbm, eij_hbm, b1_hbm, w2_hbm, b2_hbm, out_hbm,
          u0, u1, ij0, ij1, cx0, cx1, dc0, dc1, b1v, w2v, b2vv, acc,
          sg0, sg1, si0, si1, ss0, ss1):
        ub = (u0, u1)
        ijb = (ij0, ij1)
        cxb = (cx0, cx1)
        dcb = (dc0, dc1)
        sem_g = (sg0, sg1)
        sem_i = (si0, si1)
        sem_s = (ss0, ss1)

        cid = lax.axis_index("c")
        sid = lax.axis_index("s")
        wid = sid * NC + cid
        ebase = wid * EPW

        pltpu.sync_copy(b1_hbm, b1v)
        pltpu.sync_copy(w2_hbm, w2v)
        pltpu.sync_copy(b2_hbm, b2vv)

        zeros16 = jnp.zeros((16,), jnp.float32)

        def zrow(r, carry):
            for kk in range(D // 16):
                u0[r, pl.ds(kk * 16, 16)] = zeros16
            return carry

        lax.fori_loop(0, C, zrow, 0)
        rows_per_tile = NPAD // NS
        rbase = sid * rows_per_tile
        for z in range(rows_per_tile // C):
            pltpu.sync_copy(u0.at[pl.ds(0, C)], acc.at[pl.ds(rbase + z * C, C)])
        rem = rows_per_tile % C
        if rem:
            pltpu.sync_copy(u0.at[pl.ds(0, rem)],
                            acc.at[pl.ds(rbase + (rows_per_tile // C) * C, rem)])
        plsc.subcore_barrier()

        b2vec = b2vv[...]
        b2s = b2vec[0]
        zero16 = jnp.zeros((16,), jnp.float32)
        b1r = [b1v[pl.ds(kk * 16, 16)] for kk in range(D // 16)]
        w2r = [w2v[pl.ds(kk * 16, 16)] for kk in range(D // 16)]

        def fetch_idx(ci, par):
            return pltpu.async_copy(
                eij_hbm.at[:, pl.ds(ebase + ci * C, C)], ijb[par], sem_i[par])

        def copy_scatter_idx(par):
            for kk in range(C // 16):
                s_idx = ijb[par][0, pl.ds(kk * 16, 16)]
                d_idx = ijb[par][1, pl.ds(kk * 16, 16)]
                dcb[par][pl.ds(kk * 16, 16)] = d_idx
                cxb[par][pl.ds(kk * 16, 16)] = s_idx
                cxb[par][pl.ds(C + kk * 16, 16)] = d_idx + NPAD
                cxb[par][pl.ds(2 * C + kk * 16, 16)] = s_idx + 2 * NPAD

        def issue_gathers(par):
            pltpu.async_copy(U_hbm.at[cxb[par]], ub[par], sem_g[par])

        def wait_gathers(par):
            pltpu.make_async_copy(
                U_hbm.at[pl.ds(0, 3 * C)], ub[par], sem_g[par]).wait()

        def wait_idx(par):
            pltpu.make_async_copy(
                eij_hbm.at[:, pl.ds(0, C)], ijb[par], sem_i[par]).wait()

        def wait_scatter(par):
            pltpu.make_async_copy(ub[par].at[pl.ds(2 * C, C)],
                                  acc.at[pl.ds(0, C)], sem_s[par]).wait()

        def compute(par):
            u = ub[par]

            def group_body(g, gcarry):
                e0 = g * 16
                for l in range(16):
                    e = e0 + l
                    t = None
                    for kk in range(D // 16):
                        a = u[e, pl.ds(kk * 16, 16)]
                        b = u[C + e, pl.ds(kk * 16, 16)]
                        h = jnp.maximum(a + b + b1r[kk], 0.0)
                        term = h * w2r[kk]
                        t = term if t is None else t + term
                    s = jnp.sum(t) + b2s
                    av = zero16 + s
                    al_vec = 1.0 / (1.0 + jnp.exp(-av))
                    for kk in range(D // 16):
                        u[2 * C + e, pl.ds(kk * 16, 16)] = (
                            al_vec * u[2 * C + e, pl.ds(kk * 16, 16)])
                return gcarry

            lax.fori_loop(0, C // 16, group_body, 0)

        def issue_scatter(par):
            pltpu.async_copy(ub[par].at[pl.ds(2 * C, C)],
                             acc.at[dcb[par]], sem_s[par], add=True)

        fetch_idx(0, 0).wait()
        copy_scatter_idx(0)
        issue_gathers(0)
        fetch_idx(1, 1)

        wait_gathers(0)
        wait_idx(1)
        copy_scatter_idx(1)
        issue_gathers(1)
        fetch_idx(2, 0)
        compute(0)
        issue_scatter(0)

        def step(i, par):
            nxt_par = 1 - par
            wait_gathers(par)
            wait_scatter(nxt_par)
            wait_idx(nxt_par)
            copy_scatter_idx(nxt_par)
            issue_gathers(nxt_par)

            @pl.when(i + 2 < NCHUNK)
            def _():
                fetch_idx(i + 2, par)

            compute(par)
            issue_scatter(par)

        def pair_body(t, carry):
            step(2 * t + 1, 1)
            step(2 * t + 2, 0)
            return carry

        lax.fori_loop(0, (NCHUNK - 2) // 2, pair_body, 0)

        wait_gathers(1)
        wait_scatter(0)
        compute(1)
        issue_scatter(1)
        wait_scatter(1)

        plsc.subcore_barrier()
        out_rows = NPAD // NS
        pltpu.sync_copy(acc.at[pl.ds(sid * out_rows, out_rows)],
                        out_hbm.at[cid, pl.ds(sid * out_rows, out_rows)])

    return k(U, eij, b1, w2, b2v)


def _fin_body(p_ref, x_ref, bg_ref, o_ref):
    o_ref[...] = (p_ref[0] + p_ref[1] + (1.0 - RES_W) * bg_ref[...]
                  + RES_W * x_ref[...])


def _final(parts, x, bg2):
    bn = 1000
    return pl.pallas_call(
        _fin_body,
        grid=(N // bn,),
        in_specs=[pl.BlockSpec((NC, bn, D), lambda i: (0, i, 0)),
                  pl.BlockSpec((bn, D), lambda i: (i, 0)),
                  pl.BlockSpec((1, D), lambda i: (0, 0))],
        out_specs=pl.BlockSpec((bn, D), lambda i: (i, 0)),
        out_shape=jax.ShapeDtypeStruct((N, D), jnp.float32),
    )(parts, x, bg2)


def kernel(x, edge_index, W1, b1, W2, b2, Wg, bg):
    W1n = _snorm(W1)
    W2n = _snorm(W2)
    Wgn = _snorm(Wg)
    wcat = jnp.concatenate(
        [W1n[:D, :], (1.0 - RES_W) * Wgn, W1n[D:, :]], axis=1)
    xp = jnp.concatenate([x, jnp.zeros((NPAD - N, D), x.dtype)], axis=0)
    U = _precompute(xp, wcat).reshape(3 * NPAD, D)
    srcp = jnp.concatenate(
        [edge_index[0], jnp.zeros((EPADTOT - E,), jnp.int32)])
    dstp = jnp.concatenate(
        [edge_index[1], jnp.full((EPADTOT - E,), SINK, jnp.int32)])
    eij = jnp.stack([srcp, dstp])
    b2v = jnp.broadcast_to(b2.astype(jnp.float32), (16,))
    parts = _sc_edges(U, eij, b1, W2n[:, 0], b2v)
    return _final(parts, x, bg.reshape(1, D))
